# baseline (speedup 1.0000x reference)
import functools

import jax
import jax.numpy as jnp
from jax import lax
from jax.experimental import pallas as pl
from jax.experimental.pallas import tpu as pltpu
from jax.experimental.pallas import tpu_sc as plsc

B = 4096
D = 16
F = 26
UV = 1000000
SV = 100000
IN_FEAT = (F + 2) * D

NC = 2
NS = 16
NW = NC * NS
BPW = B // NW
L = 16

SP_TC = 20
SP_SC = F - SP_TC

NT = (UV + 127) // 128
DT_CH = 1024
UI_FULL = UV // DT_CH
UI_ITEMS = 2 * UI_FULL
DT_TAIL0 = UI_FULL * DT_CH
DT_T1 = 512
DT_TAIL1 = DT_TAIL0 + DT_T1
DT_REM = UV - DT_TAIL1

NT_S = (SV + 127) // 128
S_FULL = 97
S_ITEMS = SP_SC * 2 * S_FULL
S_TAIL0 = S_FULL * DT_CH
S_T1 = 640
S_TAIL1 = S_TAIL0 + S_T1
S_REM = SV - S_TAIL1
IMG_S = 2 * NT_S * 1024


@functools.cache
def _sc_tilecopy_fn():
    mesh = plsc.VectorSubcoreMesh(core_axis_name="c", subcore_axis_name="s")

    @functools.partial(
        pl.kernel,
        out_type=(
            jax.ShapeDtypeStruct((2 * NT, 8, 128), jnp.float32),
            jax.ShapeDtypeStruct((2 * NT, 8, 128), jnp.float32),
            jax.ShapeDtypeStruct((SP_SC * 2 * NT_S, 8, 128), jnp.float32),
        ),
        mesh=mesh,
        scratch_types=[
            pltpu.VMEM((2, 8, DT_CH), jnp.float32),
            pltpu.VMEM((8 * DT_REM,), jnp.float32),
            pltpu.VMEM((8, 128), jnp.float32),
            pltpu.SemaphoreType.DMA,
            pltpu.SemaphoreType.DMA,
            pltpu.SemaphoreType.DMA,
            pltpu.SemaphoreType.DMA,
            pltpu.SemaphoreType.DMA,
        ],
    )
    def _sc_tilecopy(ut2, it2, sp2t, utail, itail, sptail,
                     u_phys, i_phys, sp_phys,
                     buf, tb1, tb2, rsem0, rsem1, wsem0, wsem1, tsem):
        wid = lax.axis_index("s") * NC + lax.axis_index("c")
        rsems = (rsem0, rsem1)
        wsems = (wsem0, wsem1)
        dummy = ut2.at[pl.ds(0, 8), pl.ds(0, DT_CH)]

        def pipelined(n_items, n_iter, read_fn, write_fn):
            def valid(k):
                return (wid + k * NW) < n_items

            def slot(k, p):
                @pl.when(valid(k))
                def _():
                    pltpu.make_async_copy(dummy, buf.at[p], rsems[p]).wait()

                @pl.when(valid(k + 1))
                def _():
                    @pl.when(k + 1 >= 2)
                    def _():
                        pltpu.make_async_copy(dummy, buf.at[1 - p],
                                              wsems[1 - p]).wait()
                    read_fn(wid + (k + 1) * NW, buf.at[1 - p], rsems[1 - p])

                @pl.when(valid(k))
                def _():
                    write_fn(wid + k * NW, buf.at[p], wsems[p])

            @pl.when(valid(0))
            def _():
                read_fn(wid, buf.at[0], rsem0)

            def body(m, _):
                slot(2 * m, 0)
                slot(2 * m + 1, 1)
                return 0

            lax.fori_loop(0, (n_iter + 1) // 2, body, 0)
            @pl.when(valid(0))
            def _():
                pltpu.make_async_copy(dummy, buf.at[0], wsem0).wait()

            @pl.when(valid(1))
            def _():
                pltpu.make_async_copy(dummy, buf.at[1], wsem1).wait()

        def ui_loops(tab, phys):
            def read_fn(cid, b, rs):
                t = cid // UI_FULL
                jg = cid % UI_FULL
                pltpu.async_copy(
                    tab.at[pl.ds(8 * t, 8), pl.ds(jg * DT_CH, DT_CH)], b, rs)

            def write_fn(cid, b, ws):
                t = cid // UI_FULL
                jg = cid % UI_FULL
                for j in range(DT_CH // 128):
                    pltpu.async_copy(b.at[:, pl.ds(128 * j, 128)],
                                     phys.at[t * NT + 8 * jg + j], ws)

            pipelined(UI_ITEMS, (UI_ITEMS + NW - 1) // NW, read_fn, write_fn)

        ui_loops(ut2, u_phys)
        ui_loops(it2, i_phys)

        def sp_read(cid, b, rs):
            f = cid // (2 * S_FULL)
            rem = cid % (2 * S_FULL)
            t = rem // S_FULL
            jg = rem % S_FULL
            pltpu.async_copy(
                sp2t.at[SP_TC + f, pl.ds(8 * t, 8), pl.ds(jg * DT_CH, DT_CH)],
                b, rs)

        def sp_write(cid, b, ws):
            f = cid // (2 * S_FULL)
            rem = cid % (2 * S_FULL)
            t = rem // S_FULL
            jg = rem % S_FULL
            for j in range(DT_CH // 128):
                pltpu.async_copy(b.at[:, pl.ds(128 * j, 128)],
                                 sp_phys.at[f * 2 * NT_S + t * NT_S + 8 * jg + j],
                                 ws)

        pipelined(S_ITEMS, (S_ITEMS + NW - 1) // NW, sp_read, sp_write)

        def stage_partial_tile(tail_src, n, dst_tile):
            for r in range(8):
                for k in range(n // L):
                    tb2[r, pl.ds(L * k, L)] = tb1[pl.ds(r * n + L * k, L)]
            pltpu.async_copy(tb2, dst_tile, tsem)

        for ti, (tab, tail, phys) in enumerate(
                ((ut2, utail, u_phys), (it2, itail, i_phys))):
            for t in range(2):
                @pl.when(wid == 2 * ti + t)
                def _(tab=tab, tail=tail, phys=phys, t=t):
                    pltpu.sync_copy(
                        tab.at[pl.ds(8 * t, 8), pl.ds(DT_TAIL0, DT_T1)],
                        buf.at[0, :, pl.ds(0, DT_T1)])
                    for j in range(DT_T1 // 128):
                        pltpu.async_copy(
                            buf.at[0, :, pl.ds(128 * j, 128)],
                            phys.at[t * NT + UI_FULL * 8 + j], tsem)
                    pltpu.sync_copy(
                        tail.at[pl.ds(t * 8 * DT_REM, 8 * DT_REM)],
                        tb1.at[pl.ds(0, 8 * DT_REM)])
                    stage_partial_tile(tail, DT_REM, phys.at[t * NT + NT - 1])
                    pltpu.make_async_copy(
                        tab.at[pl.ds(0, 8), pl.ds(0, DT_T1)],
                        buf.at[0, :, pl.ds(0, DT_T1)], tsem).wait()
                    pltpu.make_async_copy(
                        tab.at[pl.ds(0, 8), pl.ds(0, 128)], tb2, tsem).wait()

        for f in range(SP_SC):
            for t in range(2):
                @pl.when(wid == 4 + 2 * f + t)
                def _(f=f, t=t):
                    pltpu.sync_copy(
                        sp2t.at[SP_TC + f, pl.ds(8 * t, 8), pl.ds(S_TAIL0, S_T1)],
                        buf.at[1, :, pl.ds(0, S_T1)])
                    for j in range(S_T1 // 128):
                        pltpu.async_copy(
                            buf.at[1, :, pl.ds(128 * j, 128)],
                            sp_phys.at[f * 2 * NT_S + t * NT_S + S_FULL * 8 + j],
                            tsem)
                    pltpu.sync_copy(
                        sptail.at[pl.ds((2 * f + t) * 8 * S_REM, 8 * S_REM)],
                        tb1.at[pl.ds(0, 8 * S_REM)])
                    stage_partial_tile(sptail, S_REM,
                                       sp_phys.at[f * 2 * NT_S + t * NT_S + NT_S - 1])
                    pltpu.make_async_copy(
                        sp2t.at[SP_TC, pl.ds(0, 8), pl.ds(0, S_T1)],
                        buf.at[1, :, pl.ds(0, S_T1)], tsem).wait()
                    pltpu.make_async_copy(
                        sp2t.at[SP_TC, pl.ds(0, 8), pl.ds(0, 128)], tb2,
                        tsem).wait()

    return _sc_tilecopy


@functools.cache
def _sc_gather_fn():
    mesh = plsc.VectorSubcoreMesh(core_axis_name="c", subcore_axis_name="s")

    @functools.partial(
        pl.kernel,
        out_type=jax.ShapeDtypeStruct((IN_FEAT, B), jnp.float32),
        mesh=mesh,
        compiler_params=pltpu.CompilerParams(use_tc_tiling_on_sc=False),
        scratch_types=[
            pltpu.VMEM((BPW,), jnp.int32),
            pltpu.VMEM((BPW,), jnp.int32),
            pltpu.VMEM((BPW,), jnp.int32),
            pltpu.VMEM((BPW,), jnp.int32),
            pltpu.VMEM((F, BPW), jnp.int32),
            pltpu.VMEM((2 * D, BPW), jnp.int32),
            pltpu.VMEM((SP_SC * D, BPW), jnp.int32),
            pltpu.VMEM((IN_FEAT, BPW), jnp.float32),
            pltpu.SemaphoreType.DMA,
        ],
    )
    def _sc_gather(u1, i1, sp1, spp, uid, iid, sft, out_hbm,
                   uidv, iidv, uph, iph, sfv, idxb, idxs, outb, sem):
        wid = lax.axis_index("s") * NC + lax.axis_index("c")
        base = wid * BPW
        pltpu.sync_copy(uid.at[pl.ds(base, BPW)], uidv)
        pltpu.sync_copy(iid.at[pl.ds(base, BPW)], iidv)
        pltpu.sync_copy(sft.at[:, pl.ds(base, BPW)], sfv)

        def phys16(v):
            return (lax.shift_left(lax.shift_right_logical(v, 7), 10)
                    + jnp.bitwise_and(v, 127))

        def vph(ids_ref, out_ref, k, _=None):
            out_ref[pl.ds(L * k, L)] = phys16(ids_ref[pl.ds(L * k, L)])
            return 0

        lax.fori_loop(0, BPW // L, functools.partial(vph, uidv, uph), 0)
        lax.fori_loop(0, BPW // L, functools.partial(vph, iidv, iph), 0)

        def build(d, _):
            c = (d // 8) * (NT * 1024) + (d % 8) * 128
            for k in range(BPW // L):
                idxb[d, pl.ds(L * k, L)] = uph[pl.ds(L * k, L)] + c
                idxb[D + d, pl.ds(L * k, L)] = iph[pl.ds(L * k, L)] + c
            return 0

        lax.fori_loop(0, D, build, 0)

        def build_s(q, _):
            d = q % D
            c = ((q // D) * IMG_S + (d // 8) * (NT_S * 1024) + (d % 8) * 128)
            f = SP_TC + q // D
            for k in range(BPW // L):
                idxs[q, pl.ds(L * k, L)] = phys16(sfv[f, pl.ds(L * k, L)]) + c
            return 0

        lax.fori_loop(0, SP_SC * D, build_s, 0)

        def fire_u(d, _):
            pltpu.async_copy(u1.at[idxb.at[d]], outb.at[d], sem)
            return 0

        def fire_i(d, _):
            pltpu.async_copy(i1.at[idxb.at[D + d]], outb.at[D + d], sem)
            return 0

        def fire_s(r, _):
            pltpu.async_copy(sp1.at[r].at[sfv.at[r // D]], outb.at[2 * D + r], sem)
            return 0

        def fire_s2(q, _):
            pltpu.async_copy(spp.at[idxs.at[q]],
                             outb.at[2 * D + SP_TC * D + q], sem)
            return 0

        lax.fori_loop(0, D, fire_u, 0)
        lax.fori_loop(0, D, fire_i, 0)
        lax.fori_loop(0, SP_TC * D, fire_s, 0)
        lax.fori_loop(0, SP_SC * D, fire_s2, 0)

        pltpu.make_async_copy(out_hbm.at[:, pl.ds(base, BPW)], outb, sem).wait()
        pltpu.sync_copy(outb, out_hbm.at[:, pl.ds(base, BPW)])

    return _sc_gather


def _dense_t_body(xt, K, cb, W0t, b0, W1t, b1, W2t, b2, Wot, bo, Wt, out):
    x0 = xt[...]
    dn = (((1,), (0,)), ((), ()))
    dot = lax.dot_general(K[0], x0, dn, preferred_element_type=jnp.float32) + cb[0]
    x1 = x0 * dot + x0
    dot = lax.dot_general(K[1], x1, dn, preferred_element_type=jnp.float32) + cb[1]
    x2 = x0 * dot + x1
    h = jnp.maximum(lax.dot_general(W0t[...], x0, dn, preferred_element_type=jnp.float32) + b0[...], 0.0)
    h = jnp.maximum(lax.dot_general(W1t[...], h, dn, preferred_element_type=jnp.float32) + b1[...], 0.0)
    h = jnp.maximum(lax.dot_general(W2t[...], h, dn, preferred_element_type=jnp.float32) + b2[...], 0.0)
    deep = lax.dot_general(Wot[...], h, dn, preferred_element_type=jnp.float32) + bo[...]
    stack = jnp.concatenate([x2, deep], axis=0)
    logit = lax.dot_general(stack, Wt[...], (((0,), (0,)), ((), ())),
                            preferred_element_type=jnp.float32)
    out[...] = 1.0 / (1.0 + jnp.exp(-logit))


def _dense_t_call(xt, K, cb, W0t, b0, W1t, b1, W2t, b2, Wot, bo, Wt):
    BB = 512
    grid = (B // BB,)
    full = lambda *s: pl.BlockSpec(s, lambda i: (0,) * len(s))
    return pl.pallas_call(
        _dense_t_body,
        grid=grid,
        in_specs=[
            pl.BlockSpec((IN_FEAT, BB), lambda i: (0, i)),
            full(2, IN_FEAT, IN_FEAT),
            full(2, IN_FEAT, 1),
            full(2 * D, IN_FEAT),
            full(2 * D, 1),
            full(2 * D, 2 * D),
            full(2 * D, 1),
            full(2 * D, 2 * D),
            full(2 * D, 1),
            full(D, 2 * D),
            full(D, 1),
            full(IN_FEAT + D, 1),
        ],
        out_specs=pl.BlockSpec((BB, 1), lambda i: (i, 0)),
        out_shape=jax.ShapeDtypeStruct((B, 1), jnp.float32),
    )(xt, K, cb, W0t, b0, W1t, b1, W2t, b2, Wot, bo, Wt)


def kernel(user_ids, item_ids, sparse_features, user_table, item_table,
           sparse_tables, kernels, cbias, W0, b0, W1, b1, W2, b2, Wo, bo, Wt):
    ut2 = user_table.T
    it2 = item_table.T
    sp2t = sparse_tables.transpose(0, 2, 1)
    utail = user_table.T[:, DT_TAIL1:].reshape(-1)
    itail = item_table.T[:, DT_TAIL1:].reshape(-1)
    sptail = sp2t[SP_TC:, :, S_TAIL1:].reshape(-1)
    u_phys, i_phys, sp_phys = _sc_tilecopy_fn()(ut2, it2, sp2t,
                                                utail, itail, sptail)
    u1 = u_phys.reshape(-1)
    i1 = i_phys.reshape(-1)
    spp = sp_phys.reshape(-1)
    sp1 = sp2t[:SP_TC].reshape(SP_TC * D, SV)
    sft = sparse_features.T.astype(jnp.int32)
    comb_t = _sc_gather_fn()(u1, i1, sp1, spp,
                             user_ids.astype(jnp.int32),
                             item_ids.astype(jnp.int32), sft)
    return _dense_t_call(
        comb_t, kernels, cbias,
        W0.T, b0.reshape(2 * D, 1), W1.T, b1.reshape(2 * D, 1),
        W2.T, b2.reshape(2 * D, 1), Wo.T, bo.reshape(D, 1), Wt)


# speedup vs baseline: 1.1490x; 1.1490x over previous
import functools

import jax
import jax.numpy as jnp
from jax import lax
from jax.experimental import pallas as pl
from jax.experimental.pallas import tpu as pltpu
from jax.experimental.pallas import tpu_sc as plsc

B = 4096
D = 16
F = 26
UV = 1000000
SV = 100000
IN_FEAT = (F + 2) * D

NC = 2
NS = 16
NW = NC * NS
BPW = B // NW
L = 16

NT = (UV + 127) // 128
DT_CH = 1024
DT_FULL = UV // DT_CH
DT_TAIL0 = DT_FULL * DT_CH
DT_T1 = 512
DT_TAIL1 = DT_TAIL0 + DT_T1
DT_REM = UV - DT_TAIL1
NGRP = DT_FULL + 1


@functools.cache
def _sc_tilecopy_fn():
    mesh = plsc.VectorSubcoreMesh(core_axis_name="c", subcore_axis_name="s")

    @functools.partial(
        pl.kernel,
        out_type=(
            jax.ShapeDtypeStruct((2 * NT, 8, 128), jnp.float32),
            jax.ShapeDtypeStruct((2 * NT, 8, 128), jnp.float32),
        ),
        mesh=mesh,
        scratch_types=[
            pltpu.VMEM((8, DT_CH), jnp.float32),
            pltpu.VMEM((8 * DT_REM,), jnp.float32),
            pltpu.VMEM((8, 128), jnp.float32),
            pltpu.SemaphoreType.DMA,
            pltpu.SemaphoreType.DMA,
        ],
    )
    def _sc_tilecopy(ut2, it2, utail, itail, u_phys, i_phys,
                     buf, tb1, tb2, wsem, tsem):
        wid = lax.axis_index("s") * NC + lax.axis_index("c")

        def table(tab, tail, phys):
            def chunk(k, _):
                cid = wid + k * NW
                t = cid // NGRP
                jg = cid % NGRP

                @pl.when((cid < 2 * NGRP) & (jg < DT_FULL))
                def _():
                    pltpu.sync_copy(
                        tab.at[pl.ds(8 * t, 8), pl.ds(jg * DT_CH, DT_CH)], buf)
                    for j in range(DT_CH // 128):
                        pltpu.async_copy(
                            buf.at[:, pl.ds(128 * j, 128)],
                            phys.at[t * NT + 8 * jg + j], wsem)
                    pltpu.make_async_copy(
                        tab.at[pl.ds(0, 8), pl.ds(0, DT_CH)], buf, wsem).wait()

                @pl.when((cid < 2 * NGRP) & (jg == DT_FULL))
                def _():
                    pltpu.sync_copy(
                        tab.at[pl.ds(8 * t, 8), pl.ds(DT_TAIL0, DT_T1)],
                        buf.at[:, pl.ds(0, DT_T1)])
                    for j in range(DT_T1 // 128):
                        pltpu.async_copy(
                            buf.at[:, pl.ds(128 * j, 128)],
                            phys.at[t * NT + DT_FULL * 8 + j], tsem)
                    pltpu.sync_copy(tail.at[pl.ds(t * 8 * DT_REM, 8 * DT_REM)], tb1)
                    for r in range(8):
                        for k in range(DT_REM // L):
                            tb2[r, pl.ds(L * k, L)] = tb1[pl.ds(r * DT_REM + L * k, L)]
                    pltpu.async_copy(tb2, phys.at[t * NT + NT - 1], tsem)
                    pltpu.make_async_copy(
                        tab.at[pl.ds(0, 8), pl.ds(0, DT_T1)],
                        buf.at[:, pl.ds(0, DT_T1)], tsem).wait()
                    pltpu.make_async_copy(
                        tab.at[pl.ds(0, 8), pl.ds(0, 128)], tb2, tsem).wait()
                return 0

            lax.fori_loop(0, (2 * NGRP + NW - 1) // NW, chunk, 0)

        table(ut2, utail, u_phys)
        table(it2, itail, i_phys)

    return _sc_tilecopy


@functools.cache
def _sc_gather_fn():
    mesh = plsc.VectorSubcoreMesh(core_axis_name="c", subcore_axis_name="s")

    @functools.partial(
        pl.kernel,
        out_type=jax.ShapeDtypeStruct((IN_FEAT, B), jnp.float32),
        mesh=mesh,
        compiler_params=pltpu.CompilerParams(use_tc_tiling_on_sc=False),
        scratch_types=[
            pltpu.VMEM((BPW,), jnp.int32),
            pltpu.VMEM((BPW,), jnp.int32),
            pltpu.VMEM((BPW,), jnp.int32),
            pltpu.VMEM((BPW,), jnp.int32),
            pltpu.VMEM((F, BPW), jnp.int32),
            pltpu.VMEM((2 * D, BPW), jnp.int32),
            pltpu.VMEM((IN_FEAT, BPW), jnp.float32),
            pltpu.SemaphoreType.DMA,
        ],
    )
    def _sc_gather(u1, i1, sp1, uid, iid, sft, out_hbm,
                   uidv, iidv, uph, iph, sfv, idxb, outb, sem):
        wid = lax.axis_index("s") * NC + lax.axis_index("c")
        base = wid * BPW
        pltpu.sync_copy(uid.at[pl.ds(base, BPW)], uidv)
        pltpu.sync_copy(iid.at[pl.ds(base, BPW)], iidv)
        pltpu.sync_copy(sft.at[:, pl.ds(base, BPW)], sfv)

        def vph(ids_ref, out_ref, k, _=None):
            v = ids_ref[pl.ds(L * k, L)]
            out_ref[pl.ds(L * k, L)] = (
                lax.shift_left(lax.shift_right_logical(v, 7), 10)
                + jnp.bitwise_and(v, 127))
            return 0

        lax.fori_loop(0, BPW // L, functools.partial(vph, uidv, uph), 0)
        lax.fori_loop(0, BPW // L, functools.partial(vph, iidv, iph), 0)

        def build(d, _):
            c = (d // 8) * (NT * 1024) + (d % 8) * 128
            for k in range(BPW // L):
                idxb[d, pl.ds(L * k, L)] = uph[pl.ds(L * k, L)] + c
                idxb[D + d, pl.ds(L * k, L)] = iph[pl.ds(L * k, L)] + c
            return 0

        lax.fori_loop(0, D, build, 0)

        def fire_u(d, _):
            pltpu.async_copy(u1.at[idxb.at[d]], outb.at[d], sem)
            return 0

        def fire_i(d, _):
            pltpu.async_copy(i1.at[idxb.at[D + d]], outb.at[D + d], sem)
            return 0

        def fire_s(r, _):
            pltpu.async_copy(sp1.at[r].at[sfv.at[r // D]], outb.at[2 * D + r], sem)
            return 0

        lax.fori_loop(0, D, fire_u, 0)
        lax.fori_loop(0, D, fire_i, 0)
        lax.fori_loop(0, F * D, fire_s, 0)

        pltpu.make_async_copy(out_hbm.at[:, pl.ds(base, BPW)], outb, sem).wait()
        pltpu.sync_copy(outb, out_hbm.at[:, pl.ds(base, BPW)])

    return _sc_gather


def _dense_t_body(xt, K, cb, W0t, b0, W1t, b1, W2t, b2, Wot, bo, Wt, out):
    x0 = xt[...]
    dn = (((1,), (0,)), ((), ()))
    dot = lax.dot_general(K[0], x0, dn, preferred_element_type=jnp.float32) + cb[0]
    x1 = x0 * dot + x0
    dot = lax.dot_general(K[1], x1, dn, preferred_element_type=jnp.float32) + cb[1]
    x2 = x0 * dot + x1
    h = jnp.maximum(lax.dot_general(W0t[...], x0, dn, preferred_element_type=jnp.float32) + b0[...], 0.0)
    h = jnp.maximum(lax.dot_general(W1t[...], h, dn, preferred_element_type=jnp.float32) + b1[...], 0.0)
    h = jnp.maximum(lax.dot_general(W2t[...], h, dn, preferred_element_type=jnp.float32) + b2[...], 0.0)
    deep = lax.dot_general(Wot[...], h, dn, preferred_element_type=jnp.float32) + bo[...]
    stack = jnp.concatenate([x2, deep], axis=0)
    logit = lax.dot_general(stack, Wt[...], (((0,), (0,)), ((), ())),
                            preferred_element_type=jnp.float32)
    out[...] = 1.0 / (1.0 + jnp.exp(-logit))


def _dense_t_call(xt, K, cb, W0t, b0, W1t, b1, W2t, b2, Wot, bo, Wt):
    BB = 512
    grid = (B // BB,)
    full = lambda *s: pl.BlockSpec(s, lambda i: (0,) * len(s))
    return pl.pallas_call(
        _dense_t_body,
        grid=grid,
        in_specs=[
            pl.BlockSpec((IN_FEAT, BB), lambda i: (0, i)),
            full(2, IN_FEAT, IN_FEAT),
            full(2, IN_FEAT, 1),
            full(2 * D, IN_FEAT),
            full(2 * D, 1),
            full(2 * D, 2 * D),
            full(2 * D, 1),
            full(2 * D, 2 * D),
            full(2 * D, 1),
            full(D, 2 * D),
            full(D, 1),
            full(IN_FEAT + D, 1),
        ],
        out_specs=pl.BlockSpec((BB, 1), lambda i: (i, 0)),
        out_shape=jax.ShapeDtypeStruct((B, 1), jnp.float32),
    )(xt, K, cb, W0t, b0, W1t, b1, W2t, b2, Wot, bo, Wt)


def kernel(user_ids, item_ids, sparse_features, user_table, item_table,
           sparse_tables, kernels, cbias, W0, b0, W1, b1, W2, b2, Wo, bo, Wt):
    ut2 = user_table.T
    it2 = item_table.T
    utail = user_table.T[:, DT_TAIL1:].reshape(-1)
    itail = item_table.T[:, DT_TAIL1:].reshape(-1)
    u_phys, i_phys = _sc_tilecopy_fn()(ut2, it2, utail, itail)
    u1 = u_phys.reshape(-1)
    i1 = i_phys.reshape(-1)
    sp1 = sparse_tables.transpose(0, 2, 1).reshape(F * D, SV)
    sft = sparse_features.T.astype(jnp.int32)
    comb_t = _sc_gather_fn()(u1, i1, sp1,
                             user_ids.astype(jnp.int32),
                             item_ids.astype(jnp.int32), sft)
    return _dense_t_call(
        comb_t, kernels, cbias,
        W0.T, b0.reshape(2 * D, 1), W1.T, b1.reshape(2 * D, 1),
        W2.T, b2.reshape(2 * D, 1), Wo.T, bo.reshape(D, 1), Wt)
